# baseline (speedup 1.0000x reference)
import functools

import jax
import jax.numpy as jnp
from jax import lax
from jax.experimental import pallas as pl
from jax.experimental.pallas import tpu as pltpu
from jax.experimental.pallas import tpu_sc as plsc

V = 1_000_000
D = 64
SC = 8.0
B = 4096 * 200
NW = 32
WPR = D // 2
NBLK = 7813
ABLK_PER_W = 245

_mesh = plsc.VectorSubcoreMesh(core_axis_name="c", subcore_axis_name="s")


@functools.partial(
    pl.kernel,
    mesh=_mesh,
    compiler_params=pltpu.CompilerParams(
        needs_layout_passes=False, use_tc_tiling_on_sc=True
    ),
    out_type=jax.ShapeDtypeStruct((V * WPR,), jnp.int32),
    scratch_types=[
        pltpu.VMEM((D, 128), jnp.float32),
        pltpu.VMEM((D, 128), jnp.float32),
        pltpu.VMEM((128 * WPR,), jnp.int32),
        pltpu.SemaphoreType.DMA,
        pltpu.SemaphoreType.DMA,
    ],
)
def _prep(wt_hbm, stg_hbm, in_a, in_b, out_v, isem_a, isem_b):
    wid = lax.axis_index("s") * 2 + lax.axis_index("c")
    iota = lax.iota(jnp.int32, 16)
    sc32 = iota * WPR
    bufs = ((in_a, isem_a), (in_b, isem_b))

    def fire(n, slot):
        blk = wid + n * NW
        in_v, isem = bufs[slot]

        def _go():
            pltpu.async_copy(
                wt_hbm.at[pl.ds(0, D), pl.ds(blk * 128, 128)], in_v, isem
            )

        pl.when(blk < NBLK)(_go)

    def drain_in(slot):
        in_v, isem = bufs[slot]
        pltpu.make_async_copy(
            wt_hbm.at[pl.ds(0, D), pl.ds(0, 128)], in_v, isem
        ).wait()

    def compute_store(n, slot):
        in_v, _ = bufs[slot]
        blk = wid + n * NW

        @plsc.parallel_loop(0, 256, 1, unroll=8)
        def _vec(q):
            j = q >> 3
            ig = q & 7
            a = in_v[2 * j, pl.ds(ig * 16, 16)]
            b = in_v[2 * j + 1, pl.ds(ig * 16, 16)]
            cw = plsc.bitcast(
                plsc.pack(a * SC, b * SC, format=plsc.PackFormat.INTERLEAVED),
                jnp.int32,
            )
            plsc.store_scatter(out_v, [ig * (16 * WPR) + sc32 + j], cw)

        pl.when(blk < NBLK - 1)(
            lambda: pltpu.sync_copy(
                out_v, stg_hbm.at[pl.ds(blk * 128 * WPR, 128 * WPR)]
            )
        )
        pl.when(blk == NBLK - 1)(
            lambda: pltpu.sync_copy(
                out_v.at[pl.ds(0, 64 * WPR)],
                stg_hbm.at[pl.ds(blk * 128 * WPR, 64 * WPR)],
            )
        )

    fire(0, 0)
    fire(1, 1)

    def pair_body(p, carry):
        for slot in range(2):
            n = 2 * p + slot
            blk = wid + n * NW

            def work(n=n, slot=slot):
                drain_in(slot)
                compute_store(n, slot)

            pl.when(blk < NBLK)(work)
            pl.when(p < ABLK_PER_W // 2 - 1)(
                lambda n=n, slot=slot: fire(n + 2, slot)
            )
        return carry

    lax.fori_loop(0, ABLK_PER_W // 2, pair_body, 0)
    n_last = ABLK_PER_W - 1
    blk_last = wid + n_last * NW

    def last():
        fire(n_last, 0)
        drain_in(0)
        compute_store(n_last, 0)

    pl.when(blk_last < NBLK)(last)


CH_B = 512
NCH_B = 200 * 8
CH_PER_W = NCH_B // NW


@functools.partial(
    pl.kernel,
    mesh=_mesh,
    compiler_params=pltpu.CompilerParams(
        needs_layout_passes=False, use_tc_tiling_on_sc=False
    ),
    out_type=jax.ShapeDtypeStruct((200, D, 4096), jnp.bfloat16),
    scratch_types=[
        pltpu.VMEM((4, 128), jnp.int32),
        pltpu.VMEM((4, 128), jnp.int32),
        pltpu.VMEM((CH_B, WPR), jnp.int32),
        pltpu.VMEM((CH_B, WPR), jnp.int32),
        pltpu.VMEM((D, CH_B), jnp.bfloat16),
        pltpu.VMEM((D, CH_B), jnp.bfloat16),
        pltpu.SemaphoreType.DMA,
        pltpu.SemaphoreType.DMA,
        pltpu.SemaphoreType.DMA,
        pltpu.SemaphoreType.DMA,
    ],
)
def _emb(
    stg_hbm, idx_hbm, out_hbm,
    idx_a, idx_b, rows_a, rows_b, out_a, out_b,
    gsem_a, gsem_b, osem_a, osem_b,
):
    wid = lax.axis_index("s") * 2 + lax.axis_index("c")
    iota = lax.iota(jnp.int32, 16)
    r32 = iota * WPR
    bufs = (
        (idx_a, rows_a, out_a, gsem_a, osem_a),
        (idx_b, rows_b, out_b, gsem_b, osem_b),
    )

    def fire(c, slot):
        idx_v, rows_v, _, gsem, _ = bufs[slot]
        s = c // 8
        tb0 = (c % 8) * 4
        q0 = s * 32 + tb0
        pltpu.sync_copy(idx_hbm.at[pl.ds(q0, 4)], idx_v)
        for k in range(4):
            pltpu.async_copy(
                stg_hbm.at[idx_v.at[k]], rows_v.at[pl.ds(k * 128, 128)], gsem
            )

    def drain_gather(slot):
        _, rows_v, _, gsem, _ = bufs[slot]
        for k in range(4):
            pltpu.make_async_copy(
                stg_hbm.at[pl.ds(0, 128)], rows_v.at[pl.ds(k * 128, 128)], gsem
            ).wait()

    def drain_store(slot):
        _, _, out_v, _, osem = bufs[slot]
        pltpu.make_async_copy(
            out_hbm.at[0, pl.ds(0, D), pl.ds(0, CH_B)], out_v, osem
        ).wait()

    def compute_store(c, slot):
        _, rows_v, out_v, _, osem = bufs[slot]

        @plsc.parallel_loop(0, 1024, 1, unroll=8)
        def _vec(q):
            d2 = q >> 5
            par = (q >> 4) & 1
            bg = q & 15
            rv = (bg * 16 + iota) * 2
            jv = jnp.full((16,), d2, jnp.int32)
            we = plsc.load_gather(rows_v, [rv, jv])
            wo = plsc.load_gather(rows_v, [rv + 1, jv])
            lo_e = we & 0xFFFF
            hi_e = wo << 16
            lo_o = (we >> 16) & 0xFFFF
            hi_o = wo & jnp.int32(-65536)
            w = jnp.where(par == 0, lo_e | hi_e, lo_o | hi_o)
            out_v[2 * d2 + par, pl.ds(bg * 32, 32)] = plsc.bitcast(
                w, jnp.bfloat16
            )

        s = c // 8
        b0 = (c % 8) * CH_B
        pltpu.async_copy(
            out_v, out_hbm.at[s, pl.ds(0, D), pl.ds(b0, CH_B)], osem
        )

    c0 = wid * CH_PER_W
    fire(c0, 0)
    fire(c0 + 1, 1)

    def pair_body(p, carry):
        for slot in range(2):
            c = c0 + 2 * p + slot
            drain_gather(slot)
            pl.when(p > 0)(lambda slot=slot: drain_store(slot))
            compute_store(c, slot)
            pl.when(p < CH_PER_W // 2 - 1)(
                lambda c=c, slot=slot: fire(c + 2, slot)
            )
        return carry

    lax.fori_loop(0, CH_PER_W // 2, pair_body, 0)
    drain_store(0)
    drain_store(1)


def kernel(input, weight):
    stg = _prep(weight.T).reshape(V, WPR)
    idx2 = input.T.reshape(200 * 32, 128)
    return _emb(stg, idx2).transpose(2, 0, 1)


# speedup vs baseline: 1.5499x; 1.5499x over previous
import functools

import jax
import jax.numpy as jnp
from jax import lax
from jax.experimental import pallas as pl
from jax.experimental.pallas import tpu as pltpu
from jax.experimental.pallas import tpu_sc as plsc

V = 1_000_000
D = 64
SC = 8.0
B = 4096 * 200
NW = 32
ROWS_PER_W = B // NW
CHUNK = 256
K = CHUNK // 128
NCHUNK = ROWS_PER_W // CHUNK
NPAIR = NCHUNK // 2

_mesh = plsc.VectorSubcoreMesh(core_axis_name="c", subcore_axis_name="s")


@functools.partial(
    pl.kernel,
    mesh=_mesh,
    compiler_params=pltpu.CompilerParams(
        needs_layout_passes=False, use_tc_tiling_on_sc=False
    ),
    out_type=jax.ShapeDtypeStruct((B, D), jnp.bfloat16),
    scratch_types=[
        pltpu.VMEM((K, 128), jnp.int32),
        pltpu.VMEM((K, 128), jnp.int32),
        pltpu.VMEM((CHUNK, D), jnp.float32),
        pltpu.VMEM((CHUNK, D), jnp.float32),
        pltpu.VMEM((CHUNK, D), jnp.bfloat16),
        pltpu.VMEM((CHUNK, D), jnp.bfloat16),
        pltpu.SemaphoreType.DMA,
        pltpu.SemaphoreType.DMA,
        pltpu.SemaphoreType.DMA,
        pltpu.SemaphoreType.DMA,
    ],
)
def _emb(
    w_hbm, idx_hbm, out_hbm,
    idx_a, idx_b, rows_a, rows_b, out_a, out_b,
    gsem_a, gsem_b, osem_a, osem_b,
):
    wid = lax.axis_index("s") * 2 + lax.axis_index("c")
    iota = lax.iota(jnp.int32, 16)
    ev = iota * 2
    bufs = (
        (idx_a, rows_a, out_a, gsem_a, osem_a),
        (idx_b, rows_b, out_b, gsem_b, osem_b),
    )

    def fire(g, slot):
        idx_v, rows_v, _, gsem, _ = bufs[slot]
        grp0 = wid * (ROWS_PER_W // 128) + g * K
        pltpu.sync_copy(idx_hbm.at[pl.ds(grp0, K)], idx_v)
        for k in range(K):
            pltpu.async_copy(
                w_hbm.at[idx_v.at[k]], rows_v.at[pl.ds(k * 128, 128)], gsem
            )

    def drain_gather(slot):
        _, rows_v, _, gsem, _ = bufs[slot]
        for k in range(K):
            pltpu.make_async_copy(
                w_hbm.at[pl.ds(0, 128)], rows_v.at[pl.ds(k * 128, 128)], gsem
            ).wait()

    def drain_store(slot):
        _, _, out_v, _, osem = bufs[slot]
        pltpu.make_async_copy(
            out_hbm.at[pl.ds(0, CHUNK)], out_v, osem
        ).wait()

    def compute_store(g, slot):
        _, rows_v, out_v, _, osem = bufs[slot]

        @plsc.parallel_loop(0, CHUNK, 1, unroll=8)
        def _row(r):
            re = jnp.full((16,), r, dtype=jnp.int32)
            for h in range(2):
                ce = ev + 32 * h
                a = plsc.load_gather(rows_v, [re, ce])
                b = plsc.load_gather(rows_v, [re, ce + 1])
                out_v[r, pl.ds(32 * h, 32)] = plsc.pack(
                    a * SC, b * SC, format=plsc.PackFormat.INTERLEAVED
                )

        row0 = wid * ROWS_PER_W + g * CHUNK
        pltpu.async_copy(out_v, out_hbm.at[pl.ds(row0, CHUNK)], osem)

    fire(0, 0)
    fire(1, 1)

    def pair_body(p, carry):
        for slot in range(2):
            g = 2 * p + slot
            drain_gather(slot)
            pl.when(p > 0)(lambda slot=slot: drain_store(slot))
            compute_store(g, slot)
            pl.when(p < NPAIR - 1)(lambda g=g, slot=slot: fire(g + 2, slot))
        return carry

    lax.fori_loop(0, NPAIR, pair_body, 0)
    drain_store(0)
    drain_store(1)


def kernel(input, weight):
    idx2 = input.reshape(B // 128, 128)
    return _emb(weight, idx2).reshape(4096, 200, D)
